# baseline (speedup 1.0000x reference)
import functools

import jax
import jax.numpy as jnp
from jax import lax
from jax.experimental import pallas as pl
from jax.experimental.pallas import tpu as pltpu
from jax.experimental.pallas import tpu_sc as plsc

VOCAB = 457
EMB = 64
VOCAB_P = 464
EMB_P = 128
SEQ_P = 512
GU = 128
G_PER_CHUNK = 2
CHUNK = GU * G_PER_CHUNK


@functools.cache
def _build(batch: int, seq: int):
    info = plsc.get_sparse_core_info()
    nw = info.num_cores * info.num_subcores
    rows_per_w = batch // nw
    assert rows_per_w * nw == batch
    chunks_per_row = SEQ_P // CHUNK

    mesh = plsc.VectorSubcoreMesh(core_axis_name="c", subcore_axis_name="s")
    out_t = jax.ShapeDtypeStruct((batch, SEQ_P, EMB_P), jnp.float32)

    @functools.partial(
        pl.kernel,
        mesh=mesh,
        out_type=[out_t, out_t],
        scratch_types=[
            pltpu.VMEM((GU,), jnp.int32),
            pltpu.VMEM((GU,), jnp.int32),
            pltpu.VMEM((GU,), jnp.int32),
            pltpu.VMEM((GU,), jnp.int32),
            pltpu.VMEM((CHUNK, EMB_P), jnp.float32),
            pltpu.VMEM((CHUNK, EMB_P), jnp.float32),
            pltpu.VMEM_SHARED((VOCAB_P, EMB_P), jnp.float32),
            pltpu.VMEM_SHARED((VOCAB_P, EMB_P), jnp.float32),
            pltpu.SemaphoreType.DMA,
            pltpu.SemaphoreType.DMA,
            pltpu.SemaphoreType.DMA,
            pltpu.SemaphoreType.DMA,
            pltpu.SemaphoreType.DMA,
            pltpu.SemaphoreType.DMA,
        ],
        compiler_params=pltpu.CompilerParams(use_tc_tiling_on_sc=True),
    )
    def k(w_seq, w_exp, seq_idx, exp_idx, o_seq, o_exp,
          i00, i01, i10, i11, rows0, rows1,
          w_seq_s, w_exp_s, si0, si1, sg0, sg1, so0, so1):
        wid = lax.axis_index("s") * info.num_cores + lax.axis_index("c")
        idx_v = [[i00, i01], [i10, i11]]
        rows_v = [rows0, rows1]
        sem_i, sem_g, sem_o = [si0, si1], [sg0, sg1], [so0, so1]

        @pl.when(lax.axis_index("s") == 0)
        def _():
            pltpu.sync_copy(w_seq, w_seq_s)
            pltpu.sync_copy(w_exp, w_exp_s)
        plsc.subcore_barrier()

        def start_idx(idx_hbm, b, c):
            irow = c * G_PER_CHUNK
            for i in range(G_PER_CHUNK):
                pltpu.async_copy(idx_hbm.at[irow + i], idx_v[b][i], sem_i[b])

        def wait_idx(idx_hbm, b):
            for i in range(G_PER_CHUNK):
                pltpu.make_async_copy(
                    idx_hbm.at[0], idx_v[b][i], sem_i[b]).wait()

        def gather_cps(w_s, b):
            return [
                pltpu.make_async_copy(
                    w_s.at[idx_v[b][i]],
                    rows_v[b].at[pl.ds(i * GU, GU)],
                    sem_g[b])
                for i in range(G_PER_CHUNK)
            ]

        def start_store(out_hbm, b, c):
            row = c // chunks_per_row
            half = c % chunks_per_row
            pltpu.async_copy(
                rows_v[b],
                out_hbm.at[row].at[pl.ds(half * CHUNK, CHUNK)],
                sem_o[b])

        def wait_store(out_hbm, b):
            pltpu.make_async_copy(
                rows_v[b], out_hbm.at[0].at[pl.ds(0, CHUNK)], sem_o[b]).wait()

        n_chunks = rows_per_w * chunks_per_row

        def do_table(idx_hbm, w_s, out_hbm):
            cbase = wid * n_chunks

            for b in range(2):
                start_idx(idx_hbm, b, cbase + b)

            def pair_body(p, carry):
                for b in range(2):
                    j = 2 * p + b
                    wait_idx(idx_hbm, b)

                    @pl.when(j >= 2)
                    def _():
                        wait_store(out_hbm, b)
                    for cp in gather_cps(w_s, b):
                        cp.start()

                for b in range(2):
                    j = 2 * p + b
                    for cp in gather_cps(w_s, b):
                        cp.wait()
                    start_store(out_hbm, b, cbase + j)

                    @pl.when(j + 2 < n_chunks)
                    def _():
                        start_idx(idx_hbm, b, cbase + j + 2)
                return carry

            lax.fori_loop(0, n_chunks // 2, pair_body, 0)

            for b in range(2):
                wait_store(out_hbm, b)

        do_table(seq_idx, w_seq_s, o_seq)
        do_table(exp_idx, w_exp_s, o_exp)

    return k


def kernel(seqs, exps, W_seq, W_exp):
    b, s = seqs.shape
    w_seq_p = jnp.pad(W_seq, ((0, VOCAB_P - VOCAB), (0, EMB_P - EMB)))
    w_exp_p = jnp.pad(W_exp, ((0, VOCAB_P - VOCAB), (0, EMB_P - EMB)))
    seq_i = jnp.pad(seqs.astype(jnp.int32), ((0, 0), (0, SEQ_P - s)))
    exp_i = jnp.pad(exps.astype(jnp.int32), ((0, 0), (0, SEQ_P - s)))
    seq_i = seq_i.reshape(b * SEQ_P // GU, GU)
    exp_i = exp_i.reshape(b * SEQ_P // GU, GU)
    o_seq, o_exp = _build(b, s)(w_seq_p, w_exp_p, seq_i, exp_i)
    return (o_seq[:, :s, :EMB], o_exp[:, :s, :EMB])
